# baseline (speedup 1.0000x reference)
import functools

import jax
import jax.numpy as jnp
from jax import lax
from jax.experimental import pallas as pl
from jax.experimental.pallas import tpu as pltpu
from jax.experimental.pallas import tpu_sc as plsc

N = 100000
D = 128
B = 1024
NS0 = 25
NS1 = 10
HID = 128
C = 50

NC, NSUB = 2, 16
NW = NC * NSUB
R2 = B * NS1 * NS0
R1 = B * NS1
ROWS_W = R2 // NW
SEGS_W = R1 // NW
CH_SEG = 8
CH_ROWS = CH_SEG * NS0
NCH = SEGS_W // CH_SEG
NBUF = 2
H1_CH = 160
VREGS = D // 16


NPW = B // NW


def _sc_body(nodes_h, s1_h, s2_h, feat_h, h0_h, h1_h, h2m_h,
             idx_v, rows_v, stage_v, h1_v, h0_v, pidx_v, sem, psem, osem):
    wid = lax.axis_index("c") * NSUB + lax.axis_index("s")

    pltpu.sync_copy(s1_h.at[pl.ds(wid * SEGS_W, SEGS_W)], pidx_v.at[pl.ds(0, SEGS_W)])
    pltpu.sync_copy(nodes_h.at[pl.ds(wid * NPW, NPW)],
                    pidx_v.at[pl.ds(SEGS_W, NPW)])
    for off in range(0, SEGS_W, 64):
        pltpu.async_copy(feat_h.at[pidx_v.at[pl.ds(off, 64)]],
                         h1_v.at[pl.ds(off, 64)], psem)
    pltpu.async_copy(feat_h.at[pidx_v.at[pl.ds(SEGS_W, NPW)]], h0_v, psem)

    pltpu.sync_copy(s2_h.at[pl.ds(wid * ROWS_W, ROWS_W)], idx_v)

    def start(ch):
        b = lax.rem(ch, NBUF)
        pltpu.async_copy(feat_h.at[idx_v.at[pl.ds(ch * CH_ROWS, 128)]],
                         rows_v.at[b, pl.ds(0, 128)], sem)
        pltpu.async_copy(feat_h.at[idx_v.at[pl.ds(ch * CH_ROWS + 128, 72)]],
                         rows_v.at[b, pl.ds(128, 72)], sem)

    def wait_chunk(b):
        pltpu.make_async_copy(feat_h.at[pl.ds(0, 128)],
                              rows_v.at[b, pl.ds(0, 128)], sem).wait()
        pltpu.make_async_copy(feat_h.at[pl.ds(0, 72)],
                              rows_v.at[b, pl.ds(128, 72)], sem).wait()

    for c in range(NBUF - 1):
        start(c)

    @pl.loop(0, NCH)
    def _chunk(ch):
        @pl.when(ch + NBUF - 1 < NCH)
        def _():
            start(ch + NBUF - 1)
        b = lax.rem(ch, NBUF)
        wait_chunk(b)

        @pl.when(ch >= NBUF)
        def _():
            pltpu.make_async_copy(stage_v.at[0], h2m_h.at[pl.ds(0, CH_SEG)],
                                  osem).wait()

        @pl.loop(0, CH_SEG, unroll=2)
        def _seg(s):
            r0 = s * NS0
            for v0 in range(0, VREGS, 4):
                accs = [rows_v[b, r0, pl.ds((v0 + v) * 16, 16)]
                        for v in range(4)]
                for r in range(1, NS0):
                    for v in range(4):
                        accs[v] = accs[v] + \
                            rows_v[b, r0 + r, pl.ds((v0 + v) * 16, 16)]
                for v in range(4):
                    stage_v[b, s, pl.ds((v0 + v) * 16, 16)] = accs[v]

        pltpu.async_copy(stage_v.at[b],
                         h2m_h.at[pl.ds(wid * SEGS_W + ch * CH_SEG, CH_SEG)],
                         osem)

    for _ in range(NBUF):
        pltpu.make_async_copy(stage_v.at[0], h2m_h.at[pl.ds(0, CH_SEG)],
                              osem).wait()

    for off in range(0, SEGS_W, 64):
        pltpu.make_async_copy(feat_h.at[pl.ds(0, 64)],
                              h1_v.at[pl.ds(off, 64)], psem).wait()
    pltpu.make_async_copy(feat_h.at[pl.ds(0, NPW)], h0_v, psem).wait()
    pltpu.sync_copy(h1_v, h1_h.at[pl.ds(wid * SEGS_W, SEGS_W)])
    pltpu.sync_copy(h0_v, h0_h.at[pl.ds(wid * NPW, NPW)])


@functools.cache
def _sc_gather_fn():
    return pl.kernel(
        _sc_body,
        out_type=(
            jax.ShapeDtypeStruct((B, D), jnp.float32),
            jax.ShapeDtypeStruct((R1, D), jnp.float32),
            jax.ShapeDtypeStruct((R1, D), jnp.float32),
        ),
        mesh=plsc.VectorSubcoreMesh(core_axis_name="c", subcore_axis_name="s",
                                    num_cores=NC, num_subcores=NSUB),
        scratch_types=(
            pltpu.VMEM((ROWS_W,), jnp.int32),
            pltpu.VMEM((NBUF, CH_ROWS, D), jnp.float32),
            pltpu.VMEM((NBUF, CH_SEG, D), jnp.float32),
            pltpu.VMEM((SEGS_W, D), jnp.float32),
            pltpu.VMEM((NPW, D), jnp.float32),
            pltpu.VMEM((SEGS_W + NPW,), jnp.int32),
            pltpu.SemaphoreType.DMA,
            pltpu.SemaphoreType.DMA,
            pltpu.SemaphoreType.DMA,
        ),
    )


GSTEPS = 8
RB = R1 // GSTEPS
GB = RB // NS1


def _mm(a, b):
    return jnp.dot(a, b, preferred_element_type=jnp.float32)


def _tc_body(h0_ref, h1_ref, h2m_ref, m10_ref, ws0_ref, wn0_ref,
             ws1_ref, wn1_ref, wp_ref, bp_ref,
             out_ref, h1m_acc, a1p_acc, a1q_acc):
    k = pl.program_id(0)
    h1b = h1_ref[...]
    h2b = h2m_ref[...]
    m10 = m10_ref[...]
    a1p = jnp.maximum(_mm(h1b, ws0_ref[...]), 0.0)
    a1q = jnp.maximum(_mm(h2b, wn0_ref[...] * (1.0 / NS0)), 0.0)
    sl = pl.ds(k * GB, GB)
    h1m_acc[sl, :] = _mm(m10, h1b)
    a1p_acc[sl, :] = _mm(m10, a1p)
    a1q_acc[sl, :] = _mm(m10, a1q)

    @pl.when(k == GSTEPS - 1)
    def _():
        a0p = jnp.maximum(_mm(h0_ref[...], ws0_ref[...]), 0.0)
        a0q = jnp.maximum(_mm(h1m_acc[...], wn0_ref[...]), 0.0)
        hl = _mm(a0p, ws1_ref[0:HID, :]) + _mm(a0q, ws1_ref[HID:, :])
        hr = (_mm(a1p_acc[...], wn1_ref[0:HID, :]) +
              _mm(a1q_acc[...], wn1_ref[HID:, :]))
        n2 = jnp.sum(hl * hl, axis=1, keepdims=True) + \
             jnp.sum(hr * hr, axis=1, keepdims=True)
        inv = 1.0 / jnp.maximum(jnp.sqrt(n2), 1e-12)
        out_ref[...] = (_mm(hl * inv, wp_ref[0:HID, :]) +
                        _mm(hr * inv, wp_ref[HID:, :]) + bp_ref[...])


def _tc_dense(h0, h1, h2m, m10, ws0, wn0, ws1, wn1, wp, bp):
    full = lambda shape: pl.BlockSpec(shape, lambda k: (0, 0))
    return pl.pallas_call(
        _tc_body,
        grid=(GSTEPS,),
        in_specs=[
            full((B, D)),
            pl.BlockSpec((RB, D), lambda k: (k, 0)),
            pl.BlockSpec((RB, D), lambda k: (k, 0)),
            full((GB, RB)),
            full((D, HID)), full((D, HID)),
            full((2 * HID, HID)), full((2 * HID, HID)),
            full((2 * HID, C)),
            full((1, C)),
        ],
        out_specs=pl.BlockSpec((B, C), lambda k: (0, 0)),
        out_shape=jax.ShapeDtypeStruct((B, C), jnp.float32),
        scratch_shapes=[
            pltpu.VMEM((B, D), jnp.float32),
            pltpu.VMEM((B, HID), jnp.float32),
            pltpu.VMEM((B, HID), jnp.float32),
        ],
        compiler_params=pltpu.CompilerParams(
            dimension_semantics=("arbitrary",)),
    )(h0, h1, h2m, m10, ws0, wn0, ws1, wn1, wp, bp)


def _group_mean_matrix():
    rows = jnp.arange(GB, dtype=jnp.int32)[:, None]
    cols = jnp.arange(RB, dtype=jnp.int32)[None, :]
    return jnp.where(cols // NS1 == rows, 1.0 / NS1, 0.0).astype(jnp.float32)


def kernel(nodes, sample1, sample2, features, W_self0, W_neigh0,
           W_self1, W_neigh1, W_pred, b_pred):
    h0, h1, h2m = _sc_gather_fn()(nodes, sample1, sample2, features)
    m10 = _group_mean_matrix()
    return _tc_dense(h0, h1, h2m, m10, W_self0, W_neigh0,
                     W_self1, W_neigh1, W_pred, b_pred.reshape(1, C))


# speedup vs baseline: 1.1368x; 1.1368x over previous
import functools

import jax
import jax.numpy as jnp
from jax import lax
from jax.experimental import pallas as pl
from jax.experimental.pallas import tpu as pltpu
from jax.experimental.pallas import tpu_sc as plsc

N = 100000
D = 128
B = 1024
NS0 = 25
NS1 = 10
HID = 128
C = 50

NC, NSUB = 2, 16
NW = NC * NSUB
R2 = B * NS1 * NS0
R1 = B * NS1
ROWS_W = R2 // NW
SEGS_W = R1 // NW
CH_SEG = 8
CH_ROWS = CH_SEG * NS0
NCH = SEGS_W // CH_SEG
NBUF = 3
NSTG = 2
H1_CH = 160
VREGS = D // 16


NPW = B // NW


def _sc_body(nodes_h, s1_h, s2_h, feat_h, h0_h, h1_h, h2m_h,
             idx_v, rows_v, stage_v, h1_v, h0_v, pidx_v, sem, psem, osem,
             qsem):
    wid = lax.axis_index("c") * NSUB + lax.axis_index("s")

    pltpu.sync_copy(s1_h.at[pl.ds(wid * SEGS_W, SEGS_W)], pidx_v.at[pl.ds(0, SEGS_W)])
    pltpu.sync_copy(nodes_h.at[pl.ds(wid * NPW, NPW)],
                    pidx_v.at[pl.ds(SEGS_W, NPW)])

    H1_PARTS = ((0, 64), (64, 64), (128, 32))

    def h1_issue(half):
        for off, n in H1_PARTS:
            pltpu.async_copy(
                feat_h.at[pidx_v.at[pl.ds(half * H1_CH + off, n)]],
                h1_v.at[pl.ds(off, n)], psem)

    def h1_drain_and_store(half):
        for off, n in H1_PARTS:
            pltpu.make_async_copy(feat_h.at[pl.ds(0, n)],
                                  h1_v.at[pl.ds(off, n)], psem).wait()
        pltpu.sync_copy(h1_v,
                        h1_h.at[pl.ds(wid * SEGS_W + half * H1_CH, H1_CH)])

    h1_issue(0)
    pltpu.async_copy(feat_h.at[pidx_v.at[pl.ds(SEGS_W, NPW)]], h0_v, qsem)

    pltpu.sync_copy(s2_h.at[pl.ds(wid * ROWS_W, ROWS_W)], idx_v)

    def start(ch):
        b = lax.rem(ch, NBUF)
        pltpu.async_copy(feat_h.at[idx_v.at[pl.ds(ch * CH_ROWS, 128)]],
                         rows_v.at[b, pl.ds(0, 128)], sem)
        pltpu.async_copy(feat_h.at[idx_v.at[pl.ds(ch * CH_ROWS + 128, 72)]],
                         rows_v.at[b, pl.ds(128, 72)], sem)

    def wait_chunk(b):
        pltpu.make_async_copy(feat_h.at[pl.ds(0, 128)],
                              rows_v.at[b, pl.ds(0, 128)], sem).wait()
        pltpu.make_async_copy(feat_h.at[pl.ds(0, 72)],
                              rows_v.at[b, pl.ds(128, 72)], sem).wait()

    for c in range(NBUF - 1):
        start(c)

    @pl.loop(0, NCH)
    def _chunk(ch):
        @pl.when(ch + NBUF - 1 < NCH)
        def _():
            start(ch + NBUF - 1)
        b = lax.rem(ch, NBUF)
        wait_chunk(b)

        @pl.when(ch == 4)
        def _():
            h1_drain_and_store(0)
            h1_issue(1)

        bs = lax.rem(ch, NSTG)

        @pl.when(ch >= NSTG)
        def _():
            pltpu.make_async_copy(stage_v.at[0], h2m_h.at[pl.ds(0, CH_SEG)],
                                  osem).wait()

        @pl.loop(0, CH_SEG, unroll=2)
        def _seg(s):
            r0 = s * NS0
            for v0 in range(0, VREGS, 4):
                accs = [rows_v[b, r0, pl.ds((v0 + v) * 16, 16)]
                        for v in range(4)]
                for r in range(1, NS0):
                    for v in range(4):
                        accs[v] = accs[v] + \
                            rows_v[b, r0 + r, pl.ds((v0 + v) * 16, 16)]
                for v in range(4):
                    stage_v[bs, s, pl.ds((v0 + v) * 16, 16)] = accs[v]

        pltpu.async_copy(stage_v.at[bs],
                         h2m_h.at[pl.ds(wid * SEGS_W + ch * CH_SEG, CH_SEG)],
                         osem)

    for _ in range(NSTG):
        pltpu.make_async_copy(stage_v.at[0], h2m_h.at[pl.ds(0, CH_SEG)],
                              osem).wait()

    h1_drain_and_store(1)
    pltpu.make_async_copy(feat_h.at[pl.ds(0, NPW)], h0_v, qsem).wait()
    pltpu.sync_copy(h0_v, h0_h.at[pl.ds(wid * NPW, NPW)])


@functools.cache
def _sc_gather_fn():
    return pl.kernel(
        _sc_body,
        out_type=(
            jax.ShapeDtypeStruct((B, D), jnp.float32),
            jax.ShapeDtypeStruct((R1, D), jnp.float32),
            jax.ShapeDtypeStruct((R1, D), jnp.float32),
        ),
        mesh=plsc.VectorSubcoreMesh(core_axis_name="c", subcore_axis_name="s",
                                    num_cores=NC, num_subcores=NSUB),
        scratch_types=(
            pltpu.VMEM((ROWS_W,), jnp.int32),
            pltpu.VMEM((NBUF, CH_ROWS, D), jnp.float32),
            pltpu.VMEM((NSTG, CH_SEG, D), jnp.float32),
            pltpu.VMEM((H1_CH, D), jnp.float32),
            pltpu.VMEM((NPW, D), jnp.float32),
            pltpu.VMEM((SEGS_W + NPW,), jnp.int32),
            pltpu.SemaphoreType.DMA,
            pltpu.SemaphoreType.DMA,
            pltpu.SemaphoreType.DMA,
            pltpu.SemaphoreType.DMA,
        ),
    )


GSTEPS = 8
RB = R1 // GSTEPS
GB = RB // NS1


def _mm(a, b):
    return jnp.dot(a, b, preferred_element_type=jnp.float32)


def _tc_body(h0_ref, h1_ref, h2m_ref, m10_ref, ws0_ref, wn0_ref,
             ws1_ref, wn1_ref, wp_ref, bp_ref,
             out_ref, h1m_acc, a1p_acc, a1q_acc):
    k = pl.program_id(0)
    h1b = h1_ref[...]
    h2b = h2m_ref[...]
    m10 = m10_ref[...]
    a1p = jnp.maximum(_mm(h1b, ws0_ref[...]), 0.0)
    a1q = jnp.maximum(_mm(h2b, wn0_ref[...] * (1.0 / NS0)), 0.0)
    sl = pl.ds(k * GB, GB)
    h1m_acc[sl, :] = _mm(m10, h1b)
    a1p_acc[sl, :] = _mm(m10, a1p)
    a1q_acc[sl, :] = _mm(m10, a1q)

    @pl.when(k == GSTEPS - 1)
    def _():
        a0p = jnp.maximum(_mm(h0_ref[...], ws0_ref[...]), 0.0)
        a0q = jnp.maximum(_mm(h1m_acc[...], wn0_ref[...]), 0.0)
        hl = _mm(a0p, ws1_ref[0:HID, :]) + _mm(a0q, ws1_ref[HID:, :])
        hr = (_mm(a1p_acc[...], wn1_ref[0:HID, :]) +
              _mm(a1q_acc[...], wn1_ref[HID:, :]))
        n2 = jnp.sum(hl * hl, axis=1, keepdims=True) + \
             jnp.sum(hr * hr, axis=1, keepdims=True)
        inv = 1.0 / jnp.maximum(jnp.sqrt(n2), 1e-12)
        out_ref[...] = (_mm(hl * inv, wp_ref[0:HID, :]) +
                        _mm(hr * inv, wp_ref[HID:, :]) + bp_ref[...])


def _tc_dense(h0, h1, h2m, m10, ws0, wn0, ws1, wn1, wp, bp):
    full = lambda shape: pl.BlockSpec(shape, lambda k: (0, 0))
    return pl.pallas_call(
        _tc_body,
        grid=(GSTEPS,),
        in_specs=[
            full((B, D)),
            pl.BlockSpec((RB, D), lambda k: (k, 0)),
            pl.BlockSpec((RB, D), lambda k: (k, 0)),
            full((GB, RB)),
            full((D, HID)), full((D, HID)),
            full((2 * HID, HID)), full((2 * HID, HID)),
            full((2 * HID, C)),
            full((1, C)),
        ],
        out_specs=pl.BlockSpec((B, C), lambda k: (0, 0)),
        out_shape=jax.ShapeDtypeStruct((B, C), jnp.float32),
        scratch_shapes=[
            pltpu.VMEM((B, D), jnp.float32),
            pltpu.VMEM((B, HID), jnp.float32),
            pltpu.VMEM((B, HID), jnp.float32),
        ],
        compiler_params=pltpu.CompilerParams(
            dimension_semantics=("arbitrary",)),
    )(h0, h1, h2m, m10, ws0, wn0, ws1, wn1, wp, bp)


def _group_mean_matrix():
    rows = jnp.arange(GB, dtype=jnp.int32)[:, None]
    cols = jnp.arange(RB, dtype=jnp.int32)[None, :]
    return jnp.where(cols // NS1 == rows, 1.0 / NS1, 0.0).astype(jnp.float32)


def kernel(nodes, sample1, sample2, features, W_self0, W_neigh0,
           W_self1, W_neigh1, W_pred, b_pred):
    h0, h1, h2m = _sc_gather_fn()(nodes, sample1, sample2, features)
    m10 = _group_mean_matrix()
    return _tc_dense(h0, h1, h2m, m10, W_self0, W_neigh0,
                     W_self1, W_neigh1, W_pred, b_pred.reshape(1, C))


# speedup vs baseline: 1.1515x; 1.0129x over previous
import functools

import jax
import jax.numpy as jnp
from jax import lax
from jax.experimental import pallas as pl
from jax.experimental.pallas import tpu as pltpu
from jax.experimental.pallas import tpu_sc as plsc

N = 100000
D = 128
B = 1024
NS0 = 25
NS1 = 10
HID = 128
C = 50

NC, NSUB = 2, 16
NW = NC * NSUB
R2 = B * NS1 * NS0
R1 = B * NS1
ROWS_W = R2 // NW
SEGS_W = R1 // NW
CH_SEG = 8
CH_ROWS = CH_SEG * NS0
NCH = SEGS_W // CH_SEG
NBUF = 3
NSTG = 2
H1_CH = 160
VREGS = D // 16


NPW = B // NW


def _sc_body(nodes_h, s1_h, s2_h, feat_h, h0_h, h1_h, h2m_h,
             idx_v, rows_v, stage_v, h1_v, h0_v, pidx_v, sem, psem, osem,
             qsem):
    wid = lax.axis_index("c") * NSUB + lax.axis_index("s")

    pltpu.sync_copy(s1_h.at[pl.ds(wid * SEGS_W, SEGS_W)], pidx_v.at[pl.ds(0, SEGS_W)])
    pltpu.sync_copy(nodes_h.at[pl.ds(wid * NPW, NPW)],
                    pidx_v.at[pl.ds(SEGS_W, NPW)])

    H1_PARTS = ((0, 64), (64, 64), (128, 32))

    def h1_issue(half):
        for off, n in H1_PARTS:
            pltpu.async_copy(
                feat_h.at[pidx_v.at[pl.ds(half * H1_CH + off, n)]],
                h1_v.at[pl.ds(off, n)], psem)

    def h1_drain_and_store(half):
        for off, n in H1_PARTS:
            pltpu.make_async_copy(feat_h.at[pl.ds(0, n)],
                                  h1_v.at[pl.ds(off, n)], psem).wait()
        pltpu.sync_copy(h1_v,
                        h1_h.at[pl.ds(wid * SEGS_W + half * H1_CH, H1_CH)])

    h1_issue(0)
    pltpu.async_copy(feat_h.at[pidx_v.at[pl.ds(SEGS_W, NPW)]], h0_v, qsem)

    pltpu.sync_copy(s2_h.at[pl.ds(wid * ROWS_W, ROWS_W)], idx_v)

    def start(ch):
        b = lax.rem(ch, NBUF)
        pltpu.async_copy(feat_h.at[idx_v.at[pl.ds(ch * CH_ROWS, 128)]],
                         rows_v.at[b, pl.ds(0, 128)], sem)
        pltpu.async_copy(feat_h.at[idx_v.at[pl.ds(ch * CH_ROWS + 128, 72)]],
                         rows_v.at[b, pl.ds(128, 72)], sem)

    def wait_chunk(b):
        pltpu.make_async_copy(feat_h.at[pl.ds(0, 128)],
                              rows_v.at[b, pl.ds(0, 128)], sem).wait()
        pltpu.make_async_copy(feat_h.at[pl.ds(0, 72)],
                              rows_v.at[b, pl.ds(128, 72)], sem).wait()

    for c in range(NBUF - 1):
        start(c)

    @pl.loop(0, NCH)
    def _chunk(ch):
        @pl.when(ch + NBUF - 1 < NCH)
        def _():
            start(ch + NBUF - 1)
        b = lax.rem(ch, NBUF)
        wait_chunk(b)

        @pl.when(ch == 4)
        def _():
            h1_drain_and_store(0)
            h1_issue(1)

        bs = lax.rem(ch, NSTG)

        @pl.when(ch >= NSTG)
        def _():
            pltpu.make_async_copy(stage_v.at[0], h2m_h.at[pl.ds(0, CH_SEG)],
                                  osem).wait()

        @pl.loop(0, CH_SEG, unroll=2)
        def _seg(s):
            r0 = s * NS0
            for v0 in range(0, VREGS, 4):
                accs = [rows_v[b, r0, pl.ds((v0 + v) * 16, 16)]
                        for v in range(4)]
                for r in range(1, NS0):
                    for v in range(4):
                        accs[v] = accs[v] + \
                            rows_v[b, r0 + r, pl.ds((v0 + v) * 16, 16)]
                for v in range(4):
                    stage_v[bs, s, pl.ds((v0 + v) * 16, 16)] = accs[v]

        pltpu.async_copy(stage_v.at[bs],
                         h2m_h.at[pl.ds(wid * SEGS_W + ch * CH_SEG, CH_SEG)],
                         osem)

    for _ in range(NSTG):
        pltpu.make_async_copy(stage_v.at[0], h2m_h.at[pl.ds(0, CH_SEG)],
                              osem).wait()

    h1_drain_and_store(1)
    pltpu.make_async_copy(feat_h.at[pl.ds(0, NPW)], h0_v, qsem).wait()
    pltpu.sync_copy(h0_v, h0_h.at[pl.ds(wid * NPW, NPW)])


@functools.cache
def _sc_gather_fn():
    return pl.kernel(
        _sc_body,
        out_type=(
            jax.ShapeDtypeStruct((B, D), jnp.float32),
            jax.ShapeDtypeStruct((R1, D), jnp.float32),
            jax.ShapeDtypeStruct((R1, D), jnp.float32),
        ),
        mesh=plsc.VectorSubcoreMesh(core_axis_name="c", subcore_axis_name="s",
                                    num_cores=NC, num_subcores=NSUB),
        scratch_types=(
            pltpu.VMEM((ROWS_W,), jnp.int32),
            pltpu.VMEM((NBUF, CH_ROWS, D), jnp.float32),
            pltpu.VMEM((NSTG, CH_SEG, D), jnp.float32),
            pltpu.VMEM((H1_CH, D), jnp.float32),
            pltpu.VMEM((NPW, D), jnp.float32),
            pltpu.VMEM((SEGS_W + NPW,), jnp.int32),
            pltpu.SemaphoreType.DMA,
            pltpu.SemaphoreType.DMA,
            pltpu.SemaphoreType.DMA,
            pltpu.SemaphoreType.DMA,
        ),
    )


GSTEPS = 8
RB = R1 // GSTEPS
GB = RB // NS1


def _mm(a, b):
    return jnp.dot(a, b, preferred_element_type=jnp.float32)


def _tc_body(h0_ref, h1_ref, h2m_ref, m10_ref, ws0_ref, wn0_ref,
             ws1_ref, wn1_ref, wp_ref, bp_ref, out_ref):
    m10 = m10_ref[...]
    ws0 = ws0_ref[...]
    wn0s = wn0_ref[...] * (1.0 / NS0)
    h1m, a1pm, a1qm = [], [], []
    for k in range(GSTEPS):
        sl = pl.ds(k * RB, RB)
        h1b = h1_ref[sl, :]
        h2b = h2m_ref[sl, :]
        a1p = jnp.maximum(_mm(h1b, ws0), 0.0)
        a1q = jnp.maximum(_mm(h2b, wn0s), 0.0)
        h1m.append(_mm(m10, h1b))
        a1pm.append(_mm(m10, a1p))
        a1qm.append(_mm(m10, a1q))
    h1m = jnp.concatenate(h1m, axis=0)
    a1pm = jnp.concatenate(a1pm, axis=0)
    a1qm = jnp.concatenate(a1qm, axis=0)
    a0p = jnp.maximum(_mm(h0_ref[...], ws0), 0.0)
    a0q = jnp.maximum(_mm(h1m, wn0_ref[...]), 0.0)
    hl = _mm(a0p, ws1_ref[0:HID, :]) + _mm(a0q, ws1_ref[HID:, :])
    hr = _mm(a1pm, wn1_ref[0:HID, :]) + _mm(a1qm, wn1_ref[HID:, :])
    n2 = jnp.sum(hl * hl, axis=1, keepdims=True) + \
         jnp.sum(hr * hr, axis=1, keepdims=True)
    inv = 1.0 / jnp.maximum(jnp.sqrt(n2), 1e-12)
    out_ref[...] = (_mm(hl * inv, wp_ref[0:HID, :]) +
                    _mm(hr * inv, wp_ref[HID:, :]) + bp_ref[...])


def _tc_dense(h0, h1, h2m, m10, ws0, wn0, ws1, wn1, wp, bp):
    return pl.pallas_call(
        _tc_body,
        out_shape=jax.ShapeDtypeStruct((B, C), jnp.float32),
    )(h0, h1, h2m, m10, ws0, wn0, ws1, wn1, wp, bp)


def _group_mean_matrix():
    rows = jnp.arange(GB, dtype=jnp.int32)[:, None]
    cols = jnp.arange(RB, dtype=jnp.int32)[None, :]
    return jnp.where(cols // NS1 == rows, 1.0 / NS1, 0.0).astype(jnp.float32)


def kernel(nodes, sample1, sample2, features, W_self0, W_neigh0,
           W_self1, W_neigh1, W_pred, b_pred):
    h0, h1, h2m = _sc_gather_fn()(nodes, sample1, sample2, features)
    m10 = _group_mean_matrix()
    return _tc_dense(h0, h1, h2m, m10, W_self0, W_neigh0,
                     W_self1, W_neigh1, W_pred, b_pred.reshape(1, C))


# speedup vs baseline: 1.1553x; 1.0033x over previous
import functools

import jax
import jax.numpy as jnp
from jax import lax
from jax.experimental import pallas as pl
from jax.experimental.pallas import tpu as pltpu
from jax.experimental.pallas import tpu_sc as plsc

N = 100000
D = 128
B = 1024
NS0 = 25
NS1 = 10
HID = 128
C = 50

NC, NSUB = 2, 16
NW = NC * NSUB
R2 = B * NS1 * NS0
R1 = B * NS1
ROWS_W = R2 // NW
SEGS_W = R1 // NW
CH_SEG = 8
CH_ROWS = CH_SEG * NS0
NCH = SEGS_W // CH_SEG
NBUF = 3
NSTG = 2
H1_CH = 160
VREGS = D // 16


NPW = B // NW


def _sc_body(nodes_h, s1_h, s2_h, feat_h, h0_h, h1_h, h2m_h,
             idx_v, rows_v, stage_v, h1_v, h0_v, pidx_v, sem, psem, osem,
             qsem):
    wid = lax.axis_index("c") * NSUB + lax.axis_index("s")

    pltpu.sync_copy(s1_h.at[pl.ds(wid * SEGS_W, SEGS_W)], pidx_v.at[pl.ds(0, SEGS_W)])
    pltpu.sync_copy(nodes_h.at[pl.ds(wid * NPW, NPW)],
                    pidx_v.at[pl.ds(SEGS_W, NPW)])

    H1_PARTS = ((0, 64), (64, 64), (128, 32))

    def h1_issue(half):
        for off, n in H1_PARTS:
            pltpu.async_copy(
                feat_h.at[pidx_v.at[pl.ds(half * H1_CH + off, n)]],
                h1_v.at[pl.ds(off, n)], psem)

    def h1_drain_and_store(half):
        for off, n in H1_PARTS:
            pltpu.make_async_copy(feat_h.at[pl.ds(0, n)],
                                  h1_v.at[pl.ds(off, n)], psem).wait()
        pltpu.sync_copy(h1_v,
                        h1_h.at[pl.ds(wid * SEGS_W + half * H1_CH, H1_CH)])

    h1_issue(0)
    pltpu.async_copy(feat_h.at[pidx_v.at[pl.ds(SEGS_W, NPW)]], h0_v, qsem)

    pltpu.sync_copy(s2_h.at[pl.ds(wid * ROWS_W, ROWS_W)], idx_v)

    def start(ch):
        b = lax.rem(ch, NBUF)
        pltpu.async_copy(feat_h.at[idx_v.at[pl.ds(ch * CH_ROWS, 128)]],
                         rows_v.at[b, pl.ds(0, 128)], sem)
        pltpu.async_copy(feat_h.at[idx_v.at[pl.ds(ch * CH_ROWS + 128, 72)]],
                         rows_v.at[b, pl.ds(128, 72)], sem)

    def wait_chunk(b):
        pltpu.make_async_copy(feat_h.at[pl.ds(0, 128)],
                              rows_v.at[b, pl.ds(0, 128)], sem).wait()
        pltpu.make_async_copy(feat_h.at[pl.ds(0, 72)],
                              rows_v.at[b, pl.ds(128, 72)], sem).wait()

    for c in range(NBUF - 1):
        start(c)

    @pl.loop(0, NCH)
    def _chunk(ch):
        @pl.when(ch + NBUF - 1 < NCH)
        def _():
            start(ch + NBUF - 1)
        b = lax.rem(ch, NBUF)
        wait_chunk(b)

        @pl.when(ch == 4)
        def _():
            h1_drain_and_store(0)
            h1_issue(1)

        bs = lax.rem(ch, NSTG)

        @pl.when(ch >= NSTG)
        def _():
            pltpu.make_async_copy(stage_v.at[0], h2m_h.at[pl.ds(0, CH_SEG)],
                                  osem).wait()

        @pl.loop(0, CH_SEG)
        def _seg(s):
            r0 = s * NS0
            for v0 in range(0, VREGS, 4):
                accs = [rows_v[b, r0, pl.ds((v0 + v) * 16, 16)]
                        for v in range(4)]
                for r in range(1, NS0):
                    for v in range(4):
                        accs[v] = accs[v] + \
                            rows_v[b, r0 + r, pl.ds((v0 + v) * 16, 16)]
                for v in range(4):
                    stage_v[bs, s, pl.ds((v0 + v) * 16, 16)] = accs[v]

        pltpu.async_copy(stage_v.at[bs],
                         h2m_h.at[pl.ds(wid * SEGS_W + ch * CH_SEG, CH_SEG)],
                         osem)

    for _ in range(NSTG):
        pltpu.make_async_copy(stage_v.at[0], h2m_h.at[pl.ds(0, CH_SEG)],
                              osem).wait()

    h1_drain_and_store(1)
    pltpu.make_async_copy(feat_h.at[pl.ds(0, NPW)], h0_v, qsem).wait()
    pltpu.sync_copy(h0_v, h0_h.at[pl.ds(wid * NPW, NPW)])


@functools.cache
def _sc_gather_fn():
    return pl.kernel(
        _sc_body,
        out_type=(
            jax.ShapeDtypeStruct((B, D), jnp.float32),
            jax.ShapeDtypeStruct((R1, D), jnp.float32),
            jax.ShapeDtypeStruct((R1, D), jnp.float32),
        ),
        mesh=plsc.VectorSubcoreMesh(core_axis_name="c", subcore_axis_name="s",
                                    num_cores=NC, num_subcores=NSUB),
        scratch_types=(
            pltpu.VMEM((ROWS_W,), jnp.int32),
            pltpu.VMEM((NBUF, CH_ROWS, D), jnp.float32),
            pltpu.VMEM((NSTG, CH_SEG, D), jnp.float32),
            pltpu.VMEM((H1_CH, D), jnp.float32),
            pltpu.VMEM((NPW, D), jnp.float32),
            pltpu.VMEM((SEGS_W + NPW,), jnp.int32),
            pltpu.SemaphoreType.DMA,
            pltpu.SemaphoreType.DMA,
            pltpu.SemaphoreType.DMA,
            pltpu.SemaphoreType.DMA,
        ),
    )


GSTEPS = 8
RB = R1 // GSTEPS
GB = RB // NS1


def _mm(a, b):
    return jnp.dot(a, b, preferred_element_type=jnp.float32)


def _tc_body(h0_ref, h1_ref, h2m_ref, m10_ref, ws0_ref, wn0_ref,
             ws1_ref, wn1_ref, wp_ref, bp_ref, out_ref):
    m10 = m10_ref[...]
    ws0 = ws0_ref[...]
    wn0s = wn0_ref[...] * (1.0 / NS0)
    h1m, a1pm, a1qm = [], [], []
    for k in range(GSTEPS):
        sl = pl.ds(k * RB, RB)
        h1b = h1_ref[sl, :]
        h2b = h2m_ref[sl, :]
        a1p = jnp.maximum(_mm(h1b, ws0), 0.0)
        a1q = jnp.maximum(_mm(h2b, wn0s), 0.0)
        h1m.append(_mm(m10, h1b))
        a1pm.append(_mm(m10, a1p))
        a1qm.append(_mm(m10, a1q))
    h1m = jnp.concatenate(h1m, axis=0)
    a1pm = jnp.concatenate(a1pm, axis=0)
    a1qm = jnp.concatenate(a1qm, axis=0)
    a0p = jnp.maximum(_mm(h0_ref[...], ws0), 0.0)
    a0q = jnp.maximum(_mm(h1m, wn0_ref[...]), 0.0)
    hl = _mm(a0p, ws1_ref[0:HID, :]) + _mm(a0q, ws1_ref[HID:, :])
    hr = _mm(a1pm, wn1_ref[0:HID, :]) + _mm(a1qm, wn1_ref[HID:, :])
    n2 = jnp.sum(hl * hl, axis=1, keepdims=True) + \
         jnp.sum(hr * hr, axis=1, keepdims=True)
    inv = 1.0 / jnp.maximum(jnp.sqrt(n2), 1e-12)
    out_ref[...] = (_mm(hl * inv, wp_ref[0:HID, :]) +
                    _mm(hr * inv, wp_ref[HID:, :]) + bp_ref[...])


def _tc_dense(h0, h1, h2m, m10, ws0, wn0, ws1, wn1, wp, bp):
    return pl.pallas_call(
        _tc_body,
        out_shape=jax.ShapeDtypeStruct((B, C), jnp.float32),
    )(h0, h1, h2m, m10, ws0, wn0, ws1, wn1, wp, bp)


def _group_mean_matrix():
    rows = jnp.arange(GB, dtype=jnp.int32)[:, None]
    cols = jnp.arange(RB, dtype=jnp.int32)[None, :]
    return jnp.where(cols // NS1 == rows, 1.0 / NS1, 0.0).astype(jnp.float32)


def kernel(nodes, sample1, sample2, features, W_self0, W_neigh0,
           W_self1, W_neigh1, W_pred, b_pred):
    h0, h1, h2m = _sc_gather_fn()(nodes, sample1, sample2, features)
    m10 = _group_mean_matrix()
    return _tc_dense(h0, h1, h2m, m10, W_self0, W_neigh0,
                     W_self1, W_neigh1, W_pred, b_pred.reshape(1, C))


# speedup vs baseline: 1.1570x; 1.0015x over previous
import functools

import jax
import jax.numpy as jnp
from jax import lax
from jax.experimental import pallas as pl
from jax.experimental.pallas import tpu as pltpu
from jax.experimental.pallas import tpu_sc as plsc

N = 100000
D = 128
B = 1024
NS0 = 25
NS1 = 10
HID = 128
C = 50

NC, NSUB = 2, 16
NW = NC * NSUB
R2 = B * NS1 * NS0
R1 = B * NS1
ROWS_W = R2 // NW
SEGS_W = R1 // NW
CH_SEG = 8
CH_ROWS = CH_SEG * NS0
NCH = SEGS_W // CH_SEG
NBUF = 3
NSTG = 2
H1_CH = 160
VREGS = D // 16


NPW = B // NW


def _sc_body(nodes_h, s1_h, s2_h, feat_h, h0_h, h1_h, h2m_h,
             idx_v, rows_v, stage_v, h1_v, h0_v, pidx_v, sem, psem, osem,
             qsem):
    wid = lax.axis_index("c") * NSUB + lax.axis_index("s")

    pltpu.sync_copy(s1_h.at[pl.ds(wid * SEGS_W, SEGS_W)], pidx_v.at[pl.ds(0, SEGS_W)])
    pltpu.sync_copy(nodes_h.at[pl.ds(wid * NPW, NPW)],
                    pidx_v.at[pl.ds(SEGS_W, NPW)])

    H1_PARTS = ((0, 64), (64, 64), (128, 32))

    def h1_issue(half):
        for off, n in H1_PARTS:
            pltpu.async_copy(
                feat_h.at[pidx_v.at[pl.ds(half * H1_CH + off, n)]],
                h1_v.at[pl.ds(off, n)], psem)

    def h1_drain_and_store(half):
        for off, n in H1_PARTS:
            pltpu.make_async_copy(feat_h.at[pl.ds(0, n)],
                                  h1_v.at[pl.ds(off, n)], psem).wait()
        pltpu.sync_copy(h1_v,
                        h1_h.at[pl.ds(wid * SEGS_W + half * H1_CH, H1_CH)])

    h1_issue(0)
    pltpu.async_copy(feat_h.at[pidx_v.at[pl.ds(SEGS_W, NPW)]], h0_v, qsem)

    pltpu.sync_copy(s2_h.at[pl.ds(wid * ROWS_W, ROWS_W)], idx_v)

    def start(ch):
        b = lax.rem(ch, NBUF)
        pltpu.async_copy(feat_h.at[idx_v.at[pl.ds(ch * CH_ROWS, CH_ROWS)]],
                         rows_v.at[b], sem)

    def wait_chunk(b):
        pltpu.make_async_copy(feat_h.at[pl.ds(0, CH_ROWS)],
                              rows_v.at[b], sem).wait()

    for c in range(NBUF - 1):
        start(c)

    @pl.loop(0, NCH)
    def _chunk(ch):
        @pl.when(ch + NBUF - 1 < NCH)
        def _():
            start(ch + NBUF - 1)
        b = lax.rem(ch, NBUF)
        wait_chunk(b)

        @pl.when(ch == 4)
        def _():
            h1_drain_and_store(0)
            h1_issue(1)

        bs = lax.rem(ch, NSTG)

        @pl.when(ch >= NSTG)
        def _():
            pltpu.make_async_copy(stage_v.at[0], h2m_h.at[pl.ds(0, CH_SEG)],
                                  osem).wait()

        @pl.loop(0, CH_SEG)
        def _seg(s):
            r0 = s * NS0
            for v0 in range(0, VREGS, 4):
                accs = [rows_v[b, r0, pl.ds((v0 + v) * 16, 16)]
                        for v in range(4)]
                for r in range(1, NS0):
                    for v in range(4):
                        accs[v] = accs[v] + \
                            rows_v[b, r0 + r, pl.ds((v0 + v) * 16, 16)]
                for v in range(4):
                    stage_v[bs, s, pl.ds((v0 + v) * 16, 16)] = accs[v]

        pltpu.async_copy(stage_v.at[bs],
                         h2m_h.at[pl.ds(wid * SEGS_W + ch * CH_SEG, CH_SEG)],
                         osem)

    for _ in range(NSTG):
        pltpu.make_async_copy(stage_v.at[0], h2m_h.at[pl.ds(0, CH_SEG)],
                              osem).wait()

    h1_drain_and_store(1)
    pltpu.make_async_copy(feat_h.at[pl.ds(0, NPW)], h0_v, qsem).wait()
    pltpu.sync_copy(h0_v, h0_h.at[pl.ds(wid * NPW, NPW)])


@functools.cache
def _sc_gather_fn():
    return pl.kernel(
        _sc_body,
        out_type=(
            jax.ShapeDtypeStruct((B, D), jnp.float32),
            jax.ShapeDtypeStruct((R1, D), jnp.float32),
            jax.ShapeDtypeStruct((R1, D), jnp.float32),
        ),
        mesh=plsc.VectorSubcoreMesh(core_axis_name="c", subcore_axis_name="s",
                                    num_cores=NC, num_subcores=NSUB),
        scratch_types=(
            pltpu.VMEM((ROWS_W,), jnp.int32),
            pltpu.VMEM((NBUF, CH_ROWS, D), jnp.float32),
            pltpu.VMEM((NSTG, CH_SEG, D), jnp.float32),
            pltpu.VMEM((H1_CH, D), jnp.float32),
            pltpu.VMEM((NPW, D), jnp.float32),
            pltpu.VMEM((SEGS_W + NPW,), jnp.int32),
            pltpu.SemaphoreType.DMA,
            pltpu.SemaphoreType.DMA,
            pltpu.SemaphoreType.DMA,
            pltpu.SemaphoreType.DMA,
        ),
    )


GSTEPS = 8
RB = R1 // GSTEPS
GB = RB // NS1


def _mm(a, b):
    return jnp.dot(a, b, preferred_element_type=jnp.float32)


def _tc_body(h0_ref, h1_ref, h2m_ref, m10_ref, ws0_ref, wn0_ref,
             ws1_ref, wn1_ref, wp_ref, bp_ref, out_ref):
    m10 = m10_ref[...]
    ws0 = ws0_ref[...]
    wn0s = wn0_ref[...] * (1.0 / NS0)
    h1m, a1pm, a1qm = [], [], []
    for k in range(GSTEPS):
        sl = pl.ds(k * RB, RB)
        h1b = h1_ref[sl, :]
        h2b = h2m_ref[sl, :]
        a1p = jnp.maximum(_mm(h1b, ws0), 0.0)
        a1q = jnp.maximum(_mm(h2b, wn0s), 0.0)
        h1m.append(_mm(m10, h1b))
        a1pm.append(_mm(m10, a1p))
        a1qm.append(_mm(m10, a1q))
    h1m = jnp.concatenate(h1m, axis=0)
    a1pm = jnp.concatenate(a1pm, axis=0)
    a1qm = jnp.concatenate(a1qm, axis=0)
    a0p = jnp.maximum(_mm(h0_ref[...], ws0), 0.0)
    a0q = jnp.maximum(_mm(h1m, wn0_ref[...]), 0.0)
    hl = _mm(a0p, ws1_ref[0:HID, :]) + _mm(a0q, ws1_ref[HID:, :])
    hr = _mm(a1pm, wn1_ref[0:HID, :]) + _mm(a1qm, wn1_ref[HID:, :])
    n2 = jnp.sum(hl * hl, axis=1, keepdims=True) + \
         jnp.sum(hr * hr, axis=1, keepdims=True)
    inv = 1.0 / jnp.maximum(jnp.sqrt(n2), 1e-12)
    out_ref[...] = (_mm(hl * inv, wp_ref[0:HID, :]) +
                    _mm(hr * inv, wp_ref[HID:, :]) + bp_ref[...])


def _tc_dense(h0, h1, h2m, m10, ws0, wn0, ws1, wn1, wp, bp):
    return pl.pallas_call(
        _tc_body,
        out_shape=jax.ShapeDtypeStruct((B, C), jnp.float32),
    )(h0, h1, h2m, m10, ws0, wn0, ws1, wn1, wp, bp)


def _group_mean_matrix():
    rows = jnp.arange(GB, dtype=jnp.int32)[:, None]
    cols = jnp.arange(RB, dtype=jnp.int32)[None, :]
    return jnp.where(cols // NS1 == rows, 1.0 / NS1, 0.0).astype(jnp.float32)


def kernel(nodes, sample1, sample2, features, W_self0, W_neigh0,
           W_self1, W_neigh1, W_pred, b_pred):
    h0, h1, h2m = _sc_gather_fn()(nodes, sample1, sample2, features)
    m10 = _group_mean_matrix()
    return _tc_dense(h0, h1, h2m, m10, W_self0, W_neigh0,
                     W_self1, W_neigh1, W_pred, b_pred.reshape(1, C))
